# baseline (speedup 1.0000x reference)
import jax
import jax.numpy as jnp
from jax import lax
from jax.experimental import pallas as pl
from jax.experimental.pallas import tpu as pltpu
from jax.experimental.pallas import tpu_sc as plsc

B = 16384
L = 200
D = 32
HIDDEN = 256
V = 1000000
RB = 512
Q = 250368
VCUT = 999936
TBLR = 4 * Q
NB1 = (VCUT - 3 * Q) // RB
NB2 = Q // RB - NB1

NC, NS = 2, 16
NW = NC * NS
SPW = B // NW
CS = 8
NCH = SPW // CS
SPLITS = ((0, 104), (104, 96))
RPS = len(SPLITS)
NDMA = CS * RPS
ROWS = CS * L
UR = 8
XPAD = 256


def _pack_body(i0, i1, i2, i3, o_ref):
    stacked = jnp.concatenate([i0[...], i1[...], i2[...], i3[...]], axis=0)
    o_ref[...] = stacked.astype(jnp.bfloat16).T


def _pack_fix_body(w4, i0, i1, i2, orph, o_ref):
    for c, r in enumerate((i0, i1, i2)):
        o_ref[:, 32 * c:32 * (c + 1)] = r[...].astype(jnp.bfloat16).T
    o_ref[:, 96:128] = jnp.zeros((RB, 32), jnp.bfloat16)

    @pl.when(pl.program_id(0) == 0)
    def _():
        o_ref[0:V - VCUT, 96:128] = orph[...].astype(jnp.bfloat16)


def _pack_table(wt, orph):
    w4 = pl.pallas_call(
        _pack_body,
        grid=(NB1,),
        in_specs=[pl.BlockSpec((32, RB), lambda i, c=c: (0, (c * Q) // RB + i))
                  for c in range(4)],
        out_specs=pl.BlockSpec((RB, 128), lambda i: (i, 0)),
        out_shape=jax.ShapeDtypeStruct((Q, 128), jnp.bfloat16),
    )(wt, wt, wt, wt)
    return pl.pallas_call(
        _pack_fix_body,
        grid=(NB2,),
        in_specs=[pl.BlockSpec(memory_space=pltpu.MemorySpace.HBM)]
                 + [pl.BlockSpec((32, RB),
                                 lambda i, c=c: (0, (c * Q) // RB + NB1 + i))
                    for c in range(3)]
                 + [pl.BlockSpec((V - VCUT, 32), lambda i: (0, 0))],
        out_specs=pl.BlockSpec((RB, 128), lambda i: (NB1 + i, 0)),
        out_shape=jax.ShapeDtypeStruct((Q, 128), jnp.bfloat16),
        input_output_aliases={0: 0},
    )(w4, wt, wt, wt, orph)


def _emb_body(x_hbm, tbl_hbm, out_hbm, idx0, idx1, rows0, rows1, outv,
              sem0, sem1):
    wid = lax.axis_index("s") * NC + lax.axis_index("c")
    obase = wid * SPW

    def start(g, idxv, rowsv, sem):
        pltpu.sync_copy(x_hbm.at[pl.ds(obase + g * CS, CS)], idxv)
        for s in range(CS):
            for off, ln in SPLITS:
                pltpu.async_copy(
                    tbl_hbm.at[idxv.at[s, pl.ds(off, ln)]],
                    rowsv.at[pl.ds(s * L + off, ln)], sem)

    def finish(g, idxv, rowsv, sem):
        for s in range(CS):
            for off, ln in SPLITS:
                pltpu.make_async_copy(
                    tbl_hbm.at[idxv.at[s, pl.ds(off, ln)]],
                    rowsv.at[pl.ds(s * L + off, ln)], sem).wait()
        for s in range(CS):
            def body(i, accs, s=s):
                a0, a1 = accs
                r = s * L + i * UR
                for u in range(UR):
                    ev, od = plsc.unpack(rowsv[r + u, :],
                                         format=plsc.PackFormat.INTERLEAVED)
                    a0 = a0 + ev
                    a1 = a1 + od
                return (a0, a1)
            z = jnp.zeros((16,), jnp.float32)
            a0, a1 = lax.fori_loop(0, L // UR, body, (z, z))
            outv[s, pl.ds(0, 16)] = a0
            outv[s, pl.ds(16, 16)] = a1
        pltpu.sync_copy(outv, out_hbm.at[pl.ds(obase + g * CS, CS)])

    start(0, idx0, rows0, sem0)
    start(1, idx1, rows1, sem1)

    def loop_body(t, carry):
        g = 2 * t
        finish(g, idx0, rows0, sem0)
        start(g + 2, idx0, rows0, sem0)
        finish(g + 1, idx1, rows1, sem1)
        start(g + 3, idx1, rows1, sem1)
        return carry

    lax.fori_loop(0, (NCH - 2) // 2, loop_body, 0)
    finish(NCH - 2, idx0, rows0, sem0)
    finish(NCH - 1, idx1, rows1, sem1)


def _embed_sum(xr, tbl):
    f = pl.kernel(
        _emb_body,
        out_type=jax.ShapeDtypeStruct((B, D), jnp.float32),
        mesh=plsc.VectorSubcoreMesh(core_axis_name="c", subcore_axis_name="s"),
        scratch_types=[
            pltpu.VMEM((CS, XPAD), jnp.int32),
            pltpu.VMEM((CS, XPAD), jnp.int32),
            pltpu.VMEM((ROWS, D), jnp.bfloat16),
            pltpu.VMEM((ROWS, D), jnp.bfloat16),
            pltpu.VMEM((CS, D), jnp.float32),
            pltpu.SemaphoreType.DMA,
            pltpu.SemaphoreType.DMA,
        ],
        compiler_params=pltpu.CompilerParams(use_tc_tiling_on_sc=False,
                                             needs_layout_passes=False),
    )
    return f(xr, tbl)


def _mlp_body(s_ref, w1t_ref, b1_ref, w2t_ref, b2_ref, o_ref):
    h = jnp.dot(s_ref[...], w1t_ref[...], preferred_element_type=jnp.float32)
    h = jnp.maximum(h + b1_ref[...], 0.0)
    o_ref[...] = (jnp.dot(h, w2t_ref[...], preferred_element_type=jnp.float32)
                  + b2_ref[...])


def _mlp(s, W1, b1, W2, b2):
    BM = 2048
    return pl.pallas_call(
        _mlp_body,
        grid=(B // BM,),
        in_specs=[
            pl.BlockSpec((BM, D), lambda i: (i, 0)),
            pl.BlockSpec((D, HIDDEN), lambda i: (0, 0)),
            pl.BlockSpec((1, HIDDEN), lambda i: (0, 0)),
            pl.BlockSpec((HIDDEN, 1), lambda i: (0, 0)),
            pl.BlockSpec((1, 1), lambda i: (0, 0)),
        ],
        out_specs=pl.BlockSpec((BM, 1), lambda i: (i, 0)),
        out_shape=jax.ShapeDtypeStruct((B, 1), jnp.float32),
    )(s, W1.T, b1.reshape(1, HIDDEN), W2.T, b2.reshape(1, 1))


def kernel(x, weight, W1, b1, W2, b2):
    wt = weight.T
    orph = weight[VCUT:V, :]
    w4 = _pack_table(wt, orph)
    tbl = w4.reshape(TBLR, D)
    v = x.astype(jnp.int32)
    xj = jnp.where(v < VCUT,
                   4 * (v % Q) + v // Q,
                   4 * (v - VCUT + (VCUT - 3 * Q)) + 3)
    xr = jnp.pad(xj, ((0, 0), (0, XPAD - L)))
    s = _embed_sum(xr, tbl)
    perm = jnp.arange(D).reshape(16, 2).T.reshape(-1)
    return _mlp(s, W1[:, perm], b1, W2, b2)


# speedup vs baseline: 1.3094x; 1.3094x over previous
import jax
import jax.numpy as jnp
from jax import lax
from jax.experimental import pallas as pl
from jax.experimental.pallas import tpu as pltpu
from jax.experimental.pallas import tpu_sc as plsc

B = 16384
L = 200
D = 32
HIDDEN = 256
V = 1000000
RB = 512
Q = 250368
VCUT = 999936
TBLR = 4 * Q
NB1 = (VCUT - 3 * Q) // RB
NB2 = Q // RB - NB1

NC, NS = 2, 16
NW = NC * NS
SPW = B // NW
CS = 8
NCH = SPW // CS
SPLITS = ((0, 104), (104, 96))
RPS = len(SPLITS)
NDMA = CS * RPS
ROWS = CS * L
UR = 8
XPAD = 256


def _pack_body(i0, i1, i2, i3, o_ref):
    stacked = jnp.concatenate([i0[...], i1[...], i2[...], i3[...]], axis=0)
    o_ref[...] = stacked.T


def _pack_fix_body(w4, i0, i1, i2, orph, o_ref):
    for c, r in enumerate((i0, i1, i2)):
        o_ref[:, 32 * c:32 * (c + 1)] = r[...].T
    o_ref[:, 96:128] = jnp.zeros((RB, 32), jnp.float32)

    @pl.when(pl.program_id(0) == 0)
    def _():
        o_ref[0:V - VCUT, 96:128] = orph[...]


def _pack_table(wt, orph):
    w4 = pl.pallas_call(
        _pack_body,
        grid=(NB1,),
        in_specs=[pl.BlockSpec((32, RB), lambda i, c=c: (0, (c * Q) // RB + i))
                  for c in range(4)],
        out_specs=pl.BlockSpec((RB, 128), lambda i: (i, 0)),
        out_shape=jax.ShapeDtypeStruct((Q, 128), jnp.float32),
    )(wt, wt, wt, wt)
    return pl.pallas_call(
        _pack_fix_body,
        grid=(NB2,),
        in_specs=[pl.BlockSpec(memory_space=pltpu.MemorySpace.HBM)]
                 + [pl.BlockSpec((32, RB),
                                 lambda i, c=c: (0, (c * Q) // RB + NB1 + i))
                    for c in range(3)]
                 + [pl.BlockSpec((V - VCUT, 32), lambda i: (0, 0))],
        out_specs=pl.BlockSpec((RB, 128), lambda i: (NB1 + i, 0)),
        out_shape=jax.ShapeDtypeStruct((Q, 128), jnp.float32),
        input_output_aliases={0: 0},
    )(w4, wt, wt, wt, orph)


def _emb_body(x_hbm, tbl_hbm, out_hbm, idx0, idx1, rows0, rows1, outv,
              sem0, sem1):
    wid = lax.axis_index("s") * NC + lax.axis_index("c")
    obase = wid * SPW

    def start(g, idxv, rowsv, sem):
        pltpu.sync_copy(x_hbm.at[pl.ds(obase + g * CS, CS)], idxv)
        for s in range(CS):
            for off, ln in SPLITS:
                pltpu.async_copy(
                    tbl_hbm.at[idxv.at[s, pl.ds(off, ln)]],
                    rowsv.at[pl.ds(s * L + off, ln)], sem)

    def finish(g, idxv, rowsv, sem):
        for s in range(CS):
            for off, ln in SPLITS:
                pltpu.make_async_copy(
                    tbl_hbm.at[idxv.at[s, pl.ds(off, ln)]],
                    rowsv.at[pl.ds(s * L + off, ln)], sem).wait()
        for s in range(CS):
            def body(i, accs, s=s):
                a0, a1 = accs
                r = s * L + i * UR
                for u in range(UR):
                    a0 = a0 + rowsv[r + u, pl.ds(0, 16)]
                    a1 = a1 + rowsv[r + u, pl.ds(16, 16)]
                return (a0, a1)
            z = jnp.zeros((16,), jnp.float32)
            a0, a1 = lax.fori_loop(0, L // UR, body, (z, z))
            outv[s, pl.ds(0, 16)] = a0
            outv[s, pl.ds(16, 16)] = a1
        pltpu.sync_copy(outv, out_hbm.at[pl.ds(obase + g * CS, CS)])

    start(0, idx0, rows0, sem0)
    start(1, idx1, rows1, sem1)

    def loop_body(t, carry):
        g = 2 * t
        finish(g, idx0, rows0, sem0)
        start(g + 2, idx0, rows0, sem0)
        finish(g + 1, idx1, rows1, sem1)
        start(g + 3, idx1, rows1, sem1)
        return carry

    lax.fori_loop(0, (NCH - 2) // 2, loop_body, 0)
    finish(NCH - 2, idx0, rows0, sem0)
    finish(NCH - 1, idx1, rows1, sem1)


def _embed_sum(xr, tbl):
    f = pl.kernel(
        _emb_body,
        out_type=jax.ShapeDtypeStruct((B, D), jnp.float32),
        mesh=plsc.VectorSubcoreMesh(core_axis_name="c", subcore_axis_name="s"),
        scratch_types=[
            pltpu.VMEM((CS, XPAD), jnp.int32),
            pltpu.VMEM((CS, XPAD), jnp.int32),
            pltpu.VMEM((ROWS, D), jnp.float32),
            pltpu.VMEM((ROWS, D), jnp.float32),
            pltpu.VMEM((CS, D), jnp.float32),
            pltpu.SemaphoreType.DMA,
            pltpu.SemaphoreType.DMA,
        ],
        compiler_params=pltpu.CompilerParams(use_tc_tiling_on_sc=False),
    )
    return f(xr, tbl)


def _mlp_body(s_ref, w1t_ref, b1_ref, w2t_ref, b2_ref, o_ref):
    h = jnp.dot(s_ref[...], w1t_ref[...], preferred_element_type=jnp.float32)
    h = jnp.maximum(h + b1_ref[...], 0.0)
    o_ref[...] = (jnp.dot(h, w2t_ref[...], preferred_element_type=jnp.float32)
                  + b2_ref[...])


def _mlp(s, W1, b1, W2, b2):
    BM = 2048
    return pl.pallas_call(
        _mlp_body,
        grid=(B // BM,),
        in_specs=[
            pl.BlockSpec((BM, D), lambda i: (i, 0)),
            pl.BlockSpec((D, HIDDEN), lambda i: (0, 0)),
            pl.BlockSpec((1, HIDDEN), lambda i: (0, 0)),
            pl.BlockSpec((HIDDEN, 1), lambda i: (0, 0)),
            pl.BlockSpec((1, 1), lambda i: (0, 0)),
        ],
        out_specs=pl.BlockSpec((BM, 1), lambda i: (i, 0)),
        out_shape=jax.ShapeDtypeStruct((B, 1), jnp.float32),
    )(s, W1.T, b1.reshape(1, HIDDEN), W2.T, b2.reshape(1, 1))


def kernel(x, weight, W1, b1, W2, b2):
    wt = weight.T
    orph = weight[VCUT:V, :]
    w4 = _pack_table(wt, orph)
    tbl = w4.reshape(TBLR, D)
    v = x.astype(jnp.int32)
    xj = jnp.where(v < VCUT,
                   4 * (v % Q) + v // Q,
                   4 * (v - VCUT + (VCUT - 3 * Q)) + 3)
    xr = jnp.pad(xj, ((0, 0), (0, XPAD - L)))
    s = _embed_sum(xr, tbl)
    return _mlp(s, W1, b1, W2, b2)
